# baseline (speedup 1.0000x reference)
import jax
import jax.numpy as jnp
from jax import lax
from jax.experimental import pallas as pl
from jax.experimental.pallas import tpu as pltpu
from jax.experimental.pallas import tpu_sc as plsc

N = 10000
E = 320000
D_IN = 128
D_OUT = 128

_NC = 2
_NS = 16
_NW = _NC * _NS
_EPT = E // _NW
_CHUNK = 128
_NCHUNK = 80
_NCHB = _NCHUNK + 8
_EPTP = _NCHB * _CHUNK
_PHASES = 2
_CPP = _NCHUNK // _PHASES
_NPAD = 10240
_RPT = _NPAD // _NS
_ZR = 64

_MM_BLK = 1000


def _mm_body(f_ref, w_ref, o_ref):
    o_ref[...] = jnp.dot(f_ref[...], w_ref[...],
                         preferred_element_type=jnp.float32)


def _merge_body(p_ref, o_ref):
    o_ref[...] = jnp.maximum(p_ref[0] + p_ref[1], 0.0)


def _sc_body(support, cols3, rows3, out, cols_v, rows_v, gbuf0, gbuf1, acc,
             sem):
    c = lax.axis_index("c")
    s = lax.axis_index("s")
    wid = c * _NS + s

    def _z(t, carry):
        gbuf0[t // 8, pl.ds((t % 8) * 16, 16)] = jnp.zeros((16,), jnp.float32)
        return carry
    lax.fori_loop(0, _ZR * 8, _z, 0)
    row0 = s * _RPT
    for k in range(_RPT // _ZR):
        pltpu.sync_copy(gbuf0.at[pl.ds(0, _ZR)],
                        acc.at[pl.ds(row0 + k * _ZR, _ZR)])
    plsc.subcore_barrier()

    def _start(j, buf):
        pltpu.async_copy(support.at[cols_v.at[j]], buf, sem)

    def _wait(j, buf):
        pltpu.make_async_copy(support.at[cols_v.at[j]], buf, sem).wait()

    def _edge(i, carry):
        j = 2 * i
        _wait(j, gbuf0)
        pltpu.sync_copy(gbuf0, acc.at[rows_v.at[j]], add=True)
        _start(j + 2, gbuf0)
        _wait(j + 1, gbuf1)
        pltpu.sync_copy(gbuf1, acc.at[rows_v.at[j + 1]], add=True)
        _start(j + 3, gbuf1)
        return carry

    for p in range(_PHASES):
        pltpu.sync_copy(cols3.at[wid, pl.ds(p * _CPP, _CPP + 8)], cols_v)
        pltpu.sync_copy(rows3.at[wid, pl.ds(p * _CPP, _CPP)], rows_v)
        _start(0, gbuf0)
        _start(1, gbuf1)
        lax.fori_loop(0, _CPP // 2, _edge, 0)
        _wait(_CPP, gbuf0)
        _wait(_CPP + 1, gbuf1)
    plsc.subcore_barrier()

    pltpu.sync_copy(acc.at[pl.ds(row0, _RPT)], out.at[c, pl.ds(row0, _RPT)])


def kernel(features, edge_index, weight):
    edge_index = edge_index.astype(jnp.int32)
    ei = edge_index.reshape(2, _NW, _EPT)
    pad_rows = jnp.full((_NW, _EPTP - _EPT), _NPAD - 1, dtype=jnp.int32)
    pad_cols = jnp.zeros((_NW, _EPTP - _EPT), dtype=jnp.int32)
    rows3 = jnp.concatenate([ei[0], pad_rows], axis=1) \
        .reshape(_NW, _NCHB, _CHUNK)
    cols3 = jnp.concatenate([ei[1], pad_cols], axis=1) \
        .reshape(_NW, _NCHB, _CHUNK)

    support = pl.pallas_call(
        _mm_body,
        grid=(N // _MM_BLK,),
        in_specs=[pl.BlockSpec((_MM_BLK, D_IN), lambda i: (i, 0)),
                  pl.BlockSpec((D_IN, D_OUT), lambda i: (0, 0))],
        out_specs=pl.BlockSpec((_MM_BLK, D_OUT), lambda i: (i, 0)),
        out_shape=jax.ShapeDtypeStruct((N, D_OUT), jnp.float32),
    )(features, weight)

    partials = pl.kernel(
        _sc_body,
        out_type=jax.ShapeDtypeStruct((_NC, _NPAD, D_OUT), jnp.float32),
        mesh=plsc.VectorSubcoreMesh(core_axis_name="c", subcore_axis_name="s"),
        scratch_types=[
            pltpu.VMEM((_CPP + 8, _CHUNK), jnp.int32),
            pltpu.VMEM((_CPP, _CHUNK), jnp.int32),
            pltpu.VMEM((_CHUNK, D_OUT), jnp.float32),
            pltpu.VMEM((_CHUNK, D_OUT), jnp.float32),
            pltpu.VMEM_SHARED((_NPAD, D_OUT), jnp.float32),
            pltpu.SemaphoreType.DMA,
        ],
    )(support, cols3, rows3)

    return pl.pallas_call(
        _merge_body,
        grid=(N // _MM_BLK,),
        in_specs=[pl.BlockSpec((_NC, _MM_BLK, D_OUT), lambda i: (0, i, 0))],
        out_specs=pl.BlockSpec((_MM_BLK, D_OUT), lambda i: (i, 0)),
        out_shape=jax.ShapeDtypeStruct((N, D_OUT), jnp.float32),
    )(partials)


# speedup vs baseline: 1.4785x; 1.4785x over previous
import jax
import jax.numpy as jnp
from jax import lax
from jax.experimental import pallas as pl
from jax.experimental.pallas import tpu as pltpu
from jax.experimental.pallas import tpu_sc as plsc

N = 10000
E = 320000
D_IN = 128
D_OUT = 128

_NC = 2
_NS = 16
_NW = _NC * _NS
_EPT = E // _NW
_CHUNK = 128
_NCHUNK = 80
_NCHB = _NCHUNK + 8
_EPTP = _NCHB * _CHUNK
_PHASES = 2
_CPP = _NCHUNK // _PHASES
_NPAD = 10240
_RPT = _NPAD // _NS
_ZR = 64

_MM_BLK = 1000


def _mm_body(f_ref, w_ref, o_ref):
    o_ref[...] = jnp.dot(f_ref[...], w_ref[...],
                         preferred_element_type=jnp.float32)


def _merge_body(p_ref, o_ref):
    o_ref[...] = jnp.maximum(p_ref[0] + p_ref[1], 0.0)


def _sc_body(support, cols3, rows3, out, cols_v, rows_v, gbuf0, gbuf1, acc,
             sem):
    c = lax.axis_index("c")
    s = lax.axis_index("s")
    wid = c * _NS + s

    def _z(t, carry):
        gbuf0[t // 8, pl.ds((t % 8) * 16, 16)] = jnp.zeros((16,), jnp.float32)
        return carry
    lax.fori_loop(0, _ZR * 8, _z, 0)
    row0 = s * _RPT
    for k in range(_RPT // _ZR):
        pltpu.sync_copy(gbuf0.at[pl.ds(0, _ZR)],
                        acc.at[pl.ds(row0 + k * _ZR, _ZR)])
    plsc.subcore_barrier()

    def _start(j, buf):
        pltpu.async_copy(support.at[cols_v.at[j]], buf, sem)

    def _wait(j, buf):
        pltpu.make_async_copy(support.at[cols_v.at[j]], buf, sem).wait()

    def _edge(j, carry):
        pltpu.async_copy(support.at[cols_v.at[j]], gbuf0, sem).wait()
        pltpu.sync_copy(gbuf0, acc.at[rows_v.at[j]], add=True)
        return carry

    for p in range(_PHASES):
        pltpu.sync_copy(cols3.at[wid, pl.ds(p * _CPP, _CPP + 8)], cols_v)
        pltpu.sync_copy(rows3.at[wid, pl.ds(p * _CPP, _CPP)], rows_v)
        lax.fori_loop(0, _CPP, _edge, 0)
    plsc.subcore_barrier()

    pltpu.sync_copy(acc.at[pl.ds(row0, _RPT)], out.at[c, pl.ds(row0, _RPT)])


def kernel(features, edge_index, weight):
    edge_index = edge_index.astype(jnp.int32)
    ei = edge_index.reshape(2, _NW, _EPT)
    pad_rows = jnp.full((_NW, _EPTP - _EPT), _NPAD - 1, dtype=jnp.int32)
    pad_cols = jnp.zeros((_NW, _EPTP - _EPT), dtype=jnp.int32)
    rows3 = jnp.concatenate([ei[0], pad_rows], axis=1) \
        .reshape(_NW, _NCHB, _CHUNK)
    cols3 = jnp.concatenate([ei[1], pad_cols], axis=1) \
        .reshape(_NW, _NCHB, _CHUNK)

    support = pl.pallas_call(
        _mm_body,
        grid=(N // _MM_BLK,),
        in_specs=[pl.BlockSpec((_MM_BLK, D_IN), lambda i: (i, 0)),
                  pl.BlockSpec((D_IN, D_OUT), lambda i: (0, 0))],
        out_specs=pl.BlockSpec((_MM_BLK, D_OUT), lambda i: (i, 0)),
        out_shape=jax.ShapeDtypeStruct((N, D_OUT), jnp.float32),
    )(features, weight)

    partials = pl.kernel(
        _sc_body,
        out_type=jax.ShapeDtypeStruct((_NC, _NPAD, D_OUT), jnp.float32),
        mesh=plsc.VectorSubcoreMesh(core_axis_name="c", subcore_axis_name="s"),
        scratch_types=[
            pltpu.VMEM((_CPP + 8, _CHUNK), jnp.int32),
            pltpu.VMEM((_CPP, _CHUNK), jnp.int32),
            pltpu.VMEM((_CHUNK, D_OUT), jnp.float32),
            pltpu.VMEM((_CHUNK, D_OUT), jnp.float32),
            pltpu.VMEM_SHARED((_NPAD, D_OUT), jnp.float32),
            pltpu.SemaphoreType.DMA,
        ],
    )(support, cols3, rows3)

    return pl.pallas_call(
        _merge_body,
        grid=(N // _MM_BLK,),
        in_specs=[pl.BlockSpec((_NC, _MM_BLK, D_OUT), lambda i: (0, i, 0))],
        out_specs=pl.BlockSpec((_MM_BLK, D_OUT), lambda i: (i, 0)),
        out_shape=jax.ShapeDtypeStruct((N, D_OUT), jnp.float32),
    )(partials)


# speedup vs baseline: 1.6350x; 1.1059x over previous
import jax
import jax.numpy as jnp
from jax import lax
from jax.experimental import pallas as pl
from jax.experimental.pallas import tpu as pltpu
from jax.experimental.pallas import tpu_sc as plsc

N = 10000
E = 320000
D_IN = 128
D_OUT = 128

_NC = 2
_NS = 16
_NW = _NC * _NS
_EPT = E // _NW
_CHUNK = 125
_NCHUNK = 80
_NCHB = _NCHUNK + 8
_PHASES = 2
_CPP = _NCHUNK // _PHASES
_NPAD = 10240
_RPT = _NPAD // _NS
_ZR = 64

_MM_BLK = 1000


def _mm_body(f_ref, w_ref, o_ref):
    o_ref[...] = jnp.dot(f_ref[...], w_ref[...],
                         preferred_element_type=jnp.float32)


def _merge_body(p_ref, o_ref):
    o_ref[...] = jnp.maximum(p_ref[0] + p_ref[1], 0.0)


def _sc_body(support, cols3, rows3, out, cols_v, rows_v, gbuf0, gbuf1, acc,
             sem):
    c = lax.axis_index("c")
    s = lax.axis_index("s")
    wid = c * _NS + s

    def _z(t, carry):
        gbuf0[t // 8, pl.ds((t % 8) * 16, 16)] = jnp.zeros((16,), jnp.float32)
        return carry
    lax.fori_loop(0, _ZR * 8, _z, 0)
    row0 = s * _RPT
    for k in range(_RPT // _ZR):
        pltpu.sync_copy(gbuf0.at[pl.ds(0, _ZR)],
                        acc.at[pl.ds(row0 + k * _ZR, _ZR)])
    plsc.subcore_barrier()

    def _start(j, buf):
        pltpu.async_copy(support.at[cols_v.at[j]], buf, sem)

    def _wait(j, buf):
        pltpu.make_async_copy(support.at[cols_v.at[j]], buf, sem).wait()

    def _edge(i, carry):
        j = 2 * i
        _wait(j, gbuf0)
        pltpu.sync_copy(gbuf0, acc.at[rows_v.at[j]], add=True)
        _start(j + 2, gbuf0)
        _wait(j + 1, gbuf1)
        pltpu.sync_copy(gbuf1, acc.at[rows_v.at[j + 1]], add=True)
        _start(j + 3, gbuf1)
        return carry

    for p in range(_PHASES):
        pltpu.sync_copy(cols3.at[wid, pl.ds(p * _CPP, _CPP + 8)], cols_v)
        pltpu.sync_copy(rows3.at[wid, pl.ds(p * _CPP, _CPP)], rows_v)
        _start(0, gbuf0)
        _start(1, gbuf1)
        lax.fori_loop(0, _CPP // 2, _edge, 0)
        _wait(_CPP, gbuf0)
        _wait(_CPP + 1, gbuf1)
    plsc.subcore_barrier()

    pltpu.sync_copy(acc.at[pl.ds(row0, _RPT)], out.at[c, pl.ds(row0, _RPT)])


def kernel(features, edge_index, weight):
    edge_index = edge_index.astype(jnp.int32)
    ei = edge_index.reshape(2, _NW, _EPT)
    rows3 = ei[0].reshape(_NW, _NCHUNK, _CHUNK)
    pad_cols = jnp.zeros((_NW, (_NCHB - _NCHUNK) * _CHUNK), dtype=jnp.int32)
    cols3 = jnp.concatenate([ei[1], pad_cols], axis=1) \
        .reshape(_NW, _NCHB, _CHUNK)

    support = pl.pallas_call(
        _mm_body,
        grid=(N // _MM_BLK,),
        in_specs=[pl.BlockSpec((_MM_BLK, D_IN), lambda i: (i, 0)),
                  pl.BlockSpec((D_IN, D_OUT), lambda i: (0, 0))],
        out_specs=pl.BlockSpec((_MM_BLK, D_OUT), lambda i: (i, 0)),
        out_shape=jax.ShapeDtypeStruct((N, D_OUT), jnp.float32),
    )(features, weight)

    partials = pl.kernel(
        _sc_body,
        out_type=jax.ShapeDtypeStruct((_NC, _NPAD, D_OUT), jnp.float32),
        mesh=plsc.VectorSubcoreMesh(core_axis_name="c", subcore_axis_name="s"),
        scratch_types=[
            pltpu.VMEM((_CPP + 8, _CHUNK), jnp.int32),
            pltpu.VMEM((_CPP, _CHUNK), jnp.int32),
            pltpu.VMEM((_CHUNK, D_OUT), jnp.float32),
            pltpu.VMEM((_CHUNK, D_OUT), jnp.float32),
            pltpu.VMEM_SHARED((_NPAD, D_OUT), jnp.float32),
            pltpu.SemaphoreType.DMA,
        ],
    )(support, cols3, rows3)

    return pl.pallas_call(
        _merge_body,
        grid=(N // _MM_BLK,),
        in_specs=[pl.BlockSpec((_NC, _MM_BLK, D_OUT), lambda i: (0, i, 0))],
        out_specs=pl.BlockSpec((_MM_BLK, D_OUT), lambda i: (i, 0)),
        out_shape=jax.ShapeDtypeStruct((N, D_OUT), jnp.float32),
    )(partials)


# speedup vs baseline: 3.7650x; 2.3027x over previous
import jax
import jax.numpy as jnp
from jax import lax
from jax.experimental import pallas as pl
from jax.experimental.pallas import tpu as pltpu
from jax.experimental.pallas import tpu_sc as plsc

N = 10000
E = 320000
D_IN = 128
D_OUT = 128

_NC = 2
_NS = 16
_NW = _NC * _NS
_CHUNK = 125
_NCHUNK = (E // _NW) // _CHUNK
_PHASES = 2
_CPP = _NCHUNK // _PHASES
_NPAD = 10240
_RPT = _NPAD // _NS
_ZR = 64

_MM_BLK = 1000


def _mm_body(f_ref, w_ref, o_ref):
    o_ref[...] = jnp.dot(f_ref[...], w_ref[...],
                         preferred_element_type=jnp.float32)


def _merge_body(p_ref, o_ref):
    o_ref[...] = jnp.maximum(p_ref[0] + p_ref[1], 0.0)


def _sc_body(support, cols3, rows3, out, cols_v, rows_v, gbuf0, gbuf1, acc,
             semg0, semg1, sems0, sems1):
    c = lax.axis_index("c")
    s = lax.axis_index("s")
    wid = c * _NS + s

    def _z(t, carry):
        gbuf0[t // 8, pl.ds((t % 8) * 16, 16)] = jnp.zeros((16,), jnp.float32)
        return carry
    lax.fori_loop(0, _ZR * 8, _z, 0)
    row0 = s * _RPT
    for k in range(_RPT // _ZR):
        pltpu.sync_copy(gbuf0.at[pl.ds(0, _ZR)],
                        acc.at[pl.ds(row0 + k * _ZR, _ZR)])
    plsc.subcore_barrier()

    def _edge(i, carry):
        j = 2 * i
        ga = pltpu.async_copy(support.at[cols_v.at[j]], gbuf0, semg0)
        gb = pltpu.async_copy(support.at[cols_v.at[j + 1]], gbuf1, semg1)
        ga.wait()
        sa = pltpu.async_copy(gbuf0, acc.at[rows_v.at[j]], sems0, add=True)
        gb.wait()
        sb = pltpu.async_copy(gbuf1, acc.at[rows_v.at[j + 1]], sems1, add=True)
        sa.wait()
        sb.wait()
        return carry

    for p in range(_PHASES):
        pltpu.sync_copy(cols3.at[wid, pl.ds(p * _CPP, _CPP)], cols_v)
        pltpu.sync_copy(rows3.at[wid, pl.ds(p * _CPP, _CPP)], rows_v)
        lax.fori_loop(0, _CPP // 2, _edge, 0)
    plsc.subcore_barrier()

    pltpu.sync_copy(acc.at[pl.ds(row0, _RPT)], out.at[c, pl.ds(row0, _RPT)])


def kernel(features, edge_index, weight):
    edge_index = edge_index.astype(jnp.int32)
    rows3 = edge_index[0].reshape(_NW, _NCHUNK, _CHUNK)
    cols3 = edge_index[1].reshape(_NW, _NCHUNK, _CHUNK)

    support = pl.pallas_call(
        _mm_body,
        grid=(N // _MM_BLK,),
        in_specs=[pl.BlockSpec((_MM_BLK, D_IN), lambda i: (i, 0)),
                  pl.BlockSpec((D_IN, D_OUT), lambda i: (0, 0))],
        out_specs=pl.BlockSpec((_MM_BLK, D_OUT), lambda i: (i, 0)),
        out_shape=jax.ShapeDtypeStruct((N, D_OUT), jnp.float32),
    )(features, weight)

    partials = pl.kernel(
        _sc_body,
        out_type=jax.ShapeDtypeStruct((_NC, _NPAD, D_OUT), jnp.float32),
        mesh=plsc.VectorSubcoreMesh(core_axis_name="c", subcore_axis_name="s"),
        scratch_types=[
            pltpu.VMEM((_CPP, _CHUNK), jnp.int32),
            pltpu.VMEM((_CPP, _CHUNK), jnp.int32),
            pltpu.VMEM((_CHUNK, D_OUT), jnp.float32),
            pltpu.VMEM((_CHUNK, D_OUT), jnp.float32),
            pltpu.VMEM_SHARED((_NPAD, D_OUT), jnp.float32),
            pltpu.SemaphoreType.DMA,
            pltpu.SemaphoreType.DMA,
            pltpu.SemaphoreType.DMA,
            pltpu.SemaphoreType.DMA,
        ],
    )(support, cols3, rows3)

    return pl.pallas_call(
        _merge_body,
        grid=(N // _MM_BLK,),
        in_specs=[pl.BlockSpec((_NC, _MM_BLK, D_OUT), lambda i: (0, i, 0))],
        out_specs=pl.BlockSpec((_MM_BLK, D_OUT), lambda i: (i, 0)),
        out_shape=jax.ShapeDtypeStruct((N, D_OUT), jnp.float32),
    )(partials)


# speedup vs baseline: 3.9247x; 1.0424x over previous
import jax
import jax.numpy as jnp
from jax import lax
from jax.experimental import pallas as pl
from jax.experimental.pallas import tpu as pltpu
from jax.experimental.pallas import tpu_sc as plsc

N = 10000
E = 320000
D_IN = 128
D_OUT = 128

_NC = 2
_NS = 16
_NW = _NC * _NS
_CHUNK = 125
_NCHUNK = (E // _NW) // _CHUNK
_PHASES = 2
_CPP = _NCHUNK // _PHASES
_NPAD = 10240
_RPT = _NPAD // _NS
_ZR = 64

_MM_BLK = 1000


def _merge_mm_body(p_ref, w_ref, o_ref):
    o_ref[...] = jnp.maximum(
        jnp.dot(p_ref[0] + p_ref[1], w_ref[...],
                preferred_element_type=jnp.float32), 0.0)


def _sc_body(support, cols3, rows3, out, cols_v, rows_v, gbuf0, gbuf1, acc,
             semg0, semg1, sems0, sems1):
    c = lax.axis_index("c")
    s = lax.axis_index("s")
    wid = c * _NS + s

    def _z(t, carry):
        gbuf0[t // 8, pl.ds((t % 8) * 16, 16)] = jnp.zeros((16,), jnp.float32)
        return carry
    lax.fori_loop(0, _ZR * 8, _z, 0)
    row0 = s * _RPT
    for k in range(_RPT // _ZR):
        pltpu.sync_copy(gbuf0.at[pl.ds(0, _ZR)],
                        acc.at[pl.ds(row0 + k * _ZR, _ZR)])
    plsc.subcore_barrier()

    def _edge(i, carry):
        j = 2 * i
        ga = pltpu.async_copy(support.at[cols_v.at[j]], gbuf0, semg0)
        gb = pltpu.async_copy(support.at[cols_v.at[j + 1]], gbuf1, semg1)
        ga.wait()
        sa = pltpu.async_copy(gbuf0, acc.at[rows_v.at[j]], sems0, add=True)
        gb.wait()
        sb = pltpu.async_copy(gbuf1, acc.at[rows_v.at[j + 1]], sems1, add=True)
        sa.wait()
        sb.wait()
        return carry

    for p in range(_PHASES):
        pltpu.sync_copy(cols3.at[wid, pl.ds(p * _CPP, _CPP)], cols_v)
        pltpu.sync_copy(rows3.at[wid, pl.ds(p * _CPP, _CPP)], rows_v)
        lax.fori_loop(0, _CPP // 2, _edge, 0)
    plsc.subcore_barrier()

    pltpu.sync_copy(acc.at[pl.ds(row0, _RPT)], out.at[c, pl.ds(row0, _RPT)])


def kernel(features, edge_index, weight):
    edge_index = edge_index.astype(jnp.int32)
    rows3 = edge_index[0].reshape(_NW, _NCHUNK, _CHUNK)
    cols3 = edge_index[1].reshape(_NW, _NCHUNK, _CHUNK)

    partials = pl.kernel(
        _sc_body,
        out_type=jax.ShapeDtypeStruct((_NC, _NPAD, D_OUT), jnp.float32),
        mesh=plsc.VectorSubcoreMesh(core_axis_name="c", subcore_axis_name="s"),
        scratch_types=[
            pltpu.VMEM((_CPP, _CHUNK), jnp.int32),
            pltpu.VMEM((_CPP, _CHUNK), jnp.int32),
            pltpu.VMEM((_CHUNK, D_OUT), jnp.float32),
            pltpu.VMEM((_CHUNK, D_OUT), jnp.float32),
            pltpu.VMEM_SHARED((_NPAD, D_OUT), jnp.float32),
            pltpu.SemaphoreType.DMA,
            pltpu.SemaphoreType.DMA,
            pltpu.SemaphoreType.DMA,
            pltpu.SemaphoreType.DMA,
        ],
    )(features, cols3, rows3)

    return pl.pallas_call(
        _merge_mm_body,
        grid=(N // _MM_BLK,),
        in_specs=[pl.BlockSpec((_NC, _MM_BLK, D_IN), lambda i: (0, i, 0)),
                  pl.BlockSpec((D_IN, D_OUT), lambda i: (0, 0))],
        out_specs=pl.BlockSpec((_MM_BLK, D_OUT), lambda i: (i, 0)),
        out_shape=jax.ShapeDtypeStruct((N, D_OUT), jnp.float32),
    )(partials, weight)


# speedup vs baseline: 3.9262x; 1.0004x over previous
import jax
import jax.numpy as jnp
from jax import lax
from jax.experimental import pallas as pl
from jax.experimental.pallas import tpu as pltpu
from jax.experimental.pallas import tpu_sc as plsc

N = 10000
E = 320000
D_IN = 128
D_OUT = 128

_NC = 2
_NS = 16
_NW = _NC * _NS
_CHUNK = 125
_NCHUNK = (E // _NW) // _CHUNK
_PHASES = 2
_CPP = _NCHUNK // _PHASES
_NPAD = 10240
_RPT = _NPAD // _NS
_ZR = 64

_MM_BLK = 1000


def _merge_mm_body(p_ref, w_ref, o_ref):
    o_ref[...] = jnp.maximum(
        jnp.dot(p_ref[0] + p_ref[1], w_ref[...],
                preferred_element_type=jnp.float32), 0.0)


def _sc_body(feats, cols3, rows3, out, cols_v, rows_v, gbuf0, gbuf1, acc,
             semg0, semg1, sems0, sems1):
    c = lax.axis_index("c")
    s = lax.axis_index("s")
    wid = c * _NS + s

    def _z(t, carry):
        gbuf0[t // 8, pl.ds((t % 8) * 16, 16)] = jnp.zeros((16,), jnp.float32)
        return carry
    lax.fori_loop(0, _ZR * 8, _z, 0)
    row0 = s * _RPT
    for k in range(_RPT // _ZR):
        pltpu.sync_copy(gbuf0.at[pl.ds(0, _ZR)],
                        acc.at[pl.ds(row0 + k * _ZR, _ZR)])
    plsc.subcore_barrier()

    def _edge(i, carry):
        j = 2 * i
        ga = pltpu.async_copy(feats.at[cols_v.at[j]], gbuf0, semg0)
        gb = pltpu.async_copy(feats.at[cols_v.at[j + 1]], gbuf1, semg1)
        ga.wait()
        sa = pltpu.async_copy(gbuf0, acc.at[rows_v.at[j]], sems0, add=True)
        gb.wait()
        sb = pltpu.async_copy(gbuf1, acc.at[rows_v.at[j + 1]], sems1, add=True)
        sa.wait()
        sb.wait()
        return carry

    for p in range(_PHASES):
        pltpu.sync_copy(cols3.at[wid, pl.ds(p * _CPP, _CPP)], cols_v)
        pltpu.sync_copy(rows3.at[wid, pl.ds(p * _CPP, _CPP)], rows_v)
        lax.fori_loop(0, _CPP // 2, _edge, 0)
    plsc.subcore_barrier()

    pltpu.sync_copy(acc.at[pl.ds(row0, _RPT)], out.at[c, pl.ds(row0, _RPT)])


def kernel(features, edge_index, weight):
    edge_index = edge_index.astype(jnp.int32)
    rows3 = edge_index[0].reshape(_NW, _NCHUNK, _CHUNK)
    cols3 = edge_index[1].reshape(_NW, _NCHUNK, _CHUNK)

    partials = pl.kernel(
        _sc_body,
        out_type=jax.ShapeDtypeStruct((_NC, _NPAD, D_OUT), jnp.float32),
        mesh=plsc.VectorSubcoreMesh(core_axis_name="c", subcore_axis_name="s"),
        scratch_types=[
            pltpu.VMEM((_CPP, _CHUNK), jnp.int32),
            pltpu.VMEM((_CPP, _CHUNK), jnp.int32),
            pltpu.VMEM((_CHUNK, D_IN), jnp.float32),
            pltpu.VMEM((_CHUNK, D_IN), jnp.float32),
            pltpu.VMEM_SHARED((_NPAD, D_IN), jnp.float32),
            pltpu.SemaphoreType.DMA,
            pltpu.SemaphoreType.DMA,
            pltpu.SemaphoreType.DMA,
            pltpu.SemaphoreType.DMA,
        ],
    )(features, cols3, rows3)

    return pl.pallas_call(
        _merge_mm_body,
        grid=(N // _MM_BLK,),
        in_specs=[pl.BlockSpec((_NC, _MM_BLK, D_IN), lambda i: (0, i, 0)),
                  pl.BlockSpec((D_IN, D_OUT), lambda i: (0, 0))],
        out_specs=pl.BlockSpec((_MM_BLK, D_OUT), lambda i: (i, 0)),
        out_shape=jax.ShapeDtypeStruct((N, D_OUT), jnp.float32),
    )(partials, weight)
